# baseline (speedup 1.0000x reference)
import jax
import jax.numpy as jnp
from jax.experimental import pallas as pl
from jax.experimental.pallas import tpu as pltpu

_BLK = 2000


def _body(t_ref, xyz_ref, rot_ref, sc_ref, op_ref, fdc_ref, fr_ref, to_ref, vel_ref,
          xyzt_ref, rotn_ref, scale_ref, opac_ref, feats_ref):
    t = t_ref[0]
    r = rot_ref[...]
    inv = jax.lax.rsqrt(jnp.maximum(jnp.sum(r * r, axis=1, keepdims=True), 1e-24))
    rotn_ref[...] = r * inv
    scale_ref[...] = jnp.exp(sc_ref[...])
    opac_ref[...] = jax.nn.sigmoid(op_ref[...])
    feats_ref[:, 0:3] = fdc_ref[...]
    feats_ref[:, 3:48] = fr_ref[...]
    xyzt_ref[...] = xyz_ref[...] + vel_ref[...] * (t - to_ref[...])


def kernel(xyz, rotation, scaling, opacity, features_dc, features_rest, time_offset, velocity, time):
    n = xyz.shape[0]
    fdc = features_dc.reshape(n, 3)
    frest = features_rest.reshape(n, 45)
    t = jnp.asarray(time, jnp.float32).reshape(1)
    blk = _BLK
    grid = (pl.cdiv(n, blk),)

    def rows(d):
        return pl.BlockSpec((blk, d), lambda i: (i, 0))

    in_specs = [
        pl.BlockSpec(memory_space=pltpu.SMEM),
        rows(3),
        rows(4),
        rows(3),
        rows(1),
        rows(3),
        rows(45),
        rows(1),
        rows(3),
    ]
    out_specs = [rows(3), rows(4), rows(3), rows(1), rows(48)]
    out_shape = [
        jax.ShapeDtypeStruct((n, 3), jnp.float32),
        jax.ShapeDtypeStruct((n, 4), jnp.float32),
        jax.ShapeDtypeStruct((n, 3), jnp.float32),
        jax.ShapeDtypeStruct((n, 1), jnp.float32),
        jax.ShapeDtypeStruct((n, 48), jnp.float32),
    ]
    xyz_t, rot, scale, opac, feats = pl.pallas_call(
        _body,
        grid=grid,
        in_specs=in_specs,
        out_specs=out_specs,
        out_shape=out_shape,
        compiler_params=pltpu.CompilerParams(
            dimension_semantics=("arbitrary",),
        ),
    )(t, xyz, rotation, scaling, opacity, fdc, frest, time_offset, velocity)
    return (xyz_t, rot, scale, opac, feats.reshape(n, 16, 3))
